# baseline (speedup 1.0000x reference)
import functools
import math

import jax
import jax.numpy as jnp
from jax import lax
from jax.experimental import pallas as pl
from jax.experimental.pallas import tpu as pltpu
from jax.experimental.pallas import tpu_sc as plsc

EMBED_W = 64
SCALE = math.sqrt(64.0)
LANES = 128
NBUF = 4


def _make_lookup(n_seq: int, n_batch: int, n_vocab: int):
    info = plsc.get_sparse_core_info()
    nc, ns = info.num_cores, info.num_subcores
    nw = nc * ns
    assert n_batch == nw * LANES
    n_outer = n_seq // NBUF
    assert n_outer * NBUF == n_seq

    mesh = plsc.VectorSubcoreMesh(core_axis_name="c", subcore_axis_name="s")

    @functools.partial(
        pl.kernel,
        mesh=mesh,
        out_type=jax.ShapeDtypeStruct((n_batch, n_seq, 2 * EMBED_W), jnp.float32),
        scratch_types=[
            pltpu.VMEM((n_seq, LANES), jnp.int32),
            pltpu.VMEM((NBUF, LANES, 2 * EMBED_W), jnp.float32),
            pltpu.SemaphoreType.DMA,
            pltpu.SemaphoreType.DMA,
            pltpu.SemaphoreType.DMA,
            pltpu.SemaphoreType.DMA,
            pltpu.SemaphoreType.DMA,
            pltpu.SemaphoreType.DMA,
            pltpu.SemaphoreType.DMA,
            pltpu.SemaphoreType.DMA,
        ],
        compiler_params=pltpu.CompilerParams(
            use_tc_tiling_on_sc=False, needs_layout_passes=False
        ),
    )
    def lookup(lutp_hbm, xt_hbm, out_hbm, idx_v, rows_v,
               g0, g1, g2, g3, w0, w1, w2, w3):
        wid = lax.axis_index("s") * nc + lax.axis_index("c")
        g_sems = [g0, g1, g2, g3]
        w_sems = [w0, w1, w2, w3]
        b0 = wid * LANES

        pltpu.sync_copy(xt_hbm.at[:, pl.ds(b0, LANES)], idx_v)

        for s in range(2):
            pltpu.async_copy(
                lutp_hbm.at[idx_v.at[s]], rows_v.at[s % NBUF], g_sems[s % NBUF]
            )

        def outer(t, carry):
            for b in range(NBUF):
                s = t * NBUF + b
                pltpu.make_async_copy(
                    lutp_hbm.at[pl.ds(0, LANES)], rows_v.at[b], g_sems[b]
                ).wait()

                pltpu.async_copy(
                    rows_v.at[b, :, pl.ds(0, EMBED_W)],
                    out_hbm.at[pl.ds(b0, LANES), s, pl.ds(0, EMBED_W)],
                    w_sems[b],
                )

                bn = (b + 2) % NBUF

                def _refill(b=b, s=s, bn=bn):
                    @pl.when(s >= 2)
                    def _drain():
                        pltpu.make_async_copy(
                            rows_v.at[bn, :, pl.ds(0, EMBED_W)],
                            out_hbm.at[pl.ds(0, LANES), 0, pl.ds(0, EMBED_W)],
                            w_sems[bn],
                        ).wait()

                    pltpu.async_copy(
                        lutp_hbm.at[idx_v.at[s + 2]], rows_v.at[bn],
                        g_sems[bn],
                    )

                if b >= 2:
                    pl.when(t < n_outer - 1)(_refill)
                else:
                    _refill()
            return carry

        lax.fori_loop(0, n_outer, outer, 0)

        for b in range(NBUF):
            pltpu.make_async_copy(
                rows_v.at[b, :, pl.ds(0, EMBED_W)],
                out_hbm.at[pl.ds(0, LANES), 0, pl.ds(0, EMBED_W)],
                w_sems[b],
            ).wait()

    return lookup


def kernel(x, lut):
    n_batch, n_seq = x.shape
    n_vocab, embed = lut.shape
    xt = x.T.astype(jnp.int32)
    lutp = jnp.pad(lut * jnp.float32(SCALE), ((0, 0), (0, embed)))
    outp = _make_lookup(n_seq, n_batch, n_vocab)(lutp, xt)
    return outp[:, :, :embed]


# speedup vs baseline: 1.3315x; 1.3315x over previous
import functools
import math

import jax
import jax.numpy as jnp
from jax import lax
from jax.experimental import pallas as pl
from jax.experimental.pallas import tpu as pltpu
from jax.experimental.pallas import tpu_sc as plsc

EMBED_W = 64
SCALE = math.sqrt(64.0)
LANES = 128
NBUF = 4


def _make_lookup(n_seq: int, n_batch: int, n_vocab: int):
    info = plsc.get_sparse_core_info()
    nc, ns = info.num_cores, info.num_subcores
    nw = nc * ns
    assert n_batch == nw * LANES
    n_outer = n_seq // NBUF
    assert n_outer * NBUF == n_seq

    mesh = plsc.VectorSubcoreMesh(core_axis_name="c", subcore_axis_name="s")

    @functools.partial(
        pl.kernel,
        mesh=mesh,
        out_type=jax.ShapeDtypeStruct((n_batch, n_seq, 2 * EMBED_W), jnp.float32),
        scratch_types=[
            pltpu.VMEM((n_seq, LANES), jnp.int32),
            pltpu.VMEM((NBUF, LANES, 2 * EMBED_W), jnp.float32),
            pltpu.SemaphoreType.DMA,
            pltpu.SemaphoreType.DMA,
            pltpu.SemaphoreType.DMA,
            pltpu.SemaphoreType.DMA,
            pltpu.SemaphoreType.DMA,
            pltpu.SemaphoreType.DMA,
            pltpu.SemaphoreType.DMA,
            pltpu.SemaphoreType.DMA,
        ],
        compiler_params=pltpu.CompilerParams(
            use_tc_tiling_on_sc=False, needs_layout_passes=False
        ),
    )
    def lookup(lutp_hbm, xt_hbm, out_hbm, idx_v, rows_v,
               g0, g1, g2, g3, w0, w1, w2, w3):
        wid = lax.axis_index("s") * nc + lax.axis_index("c")
        g_sems = [g0, g1, g2, g3]
        w_sems = [w0, w1, w2, w3]
        b0 = wid * LANES

        pltpu.sync_copy(xt_hbm.at[:, pl.ds(b0, LANES)], idx_v)

        for s in range(2):
            pltpu.async_copy(
                lutp_hbm.at[idx_v.at[s]], rows_v.at[s % NBUF], g_sems[s % NBUF]
            )

        def outer(t, carry):
            for b in range(NBUF):
                s = t * NBUF + b
                pltpu.make_async_copy(
                    lutp_hbm.at[pl.ds(0, LANES)], rows_v.at[b], g_sems[b]
                ).wait()

                pltpu.async_copy(
                    rows_v.at[b, :, pl.ds(0, EMBED_W)],
                    out_hbm.at[pl.ds(b0, LANES), s, pl.ds(0, EMBED_W)],
                    w_sems[b],
                )

                bn = (b + 2) % NBUF

                def _refill(b=b, s=s, bn=bn):
                    @pl.when(s >= 2)
                    def _drain():
                        pltpu.make_async_copy(
                            rows_v.at[bn, :, pl.ds(0, EMBED_W)],
                            out_hbm.at[pl.ds(0, LANES), 0, pl.ds(0, EMBED_W)],
                            w_sems[bn],
                        ).wait()

                    pltpu.async_copy(
                        lutp_hbm.at[idx_v.at[s + 2]], rows_v.at[bn],
                        g_sems[bn],
                    )

                if b >= 2:
                    pl.when(t < n_outer - 1)(_refill)
                else:
                    _refill()
            return carry

        lax.fori_loop(0, n_outer, outer, 0)

        for b in range(NBUF):
            pltpu.make_async_copy(
                rows_v.at[b, :, pl.ds(0, EMBED_W)],
                out_hbm.at[pl.ds(0, LANES), 0, pl.ds(0, EMBED_W)],
                w_sems[b],
            ).wait()

    return lookup


def kernel(x, lut):
    n_batch, n_seq = x.shape
    n_vocab, embed = lut.shape
    xt = x.T.astype(jnp.int32)
    lutp = jnp.pad(lut, ((0, 0), (0, embed))) * jnp.float32(SCALE)
    outp = _make_lookup(n_seq, n_batch, n_vocab)(lutp, xt)
    return outp[:, :, :embed]
